# baseline (speedup 1.0000x reference)
import jax
import jax.numpy as jnp
from jax import lax
from jax.experimental import pallas as pl
from jax.experimental.pallas import tpu as pltpu
from jax.experimental.pallas import tpu_sc as plsc
import functools

RES_F = 100000.0
EMBED = 64
NW = 32
ROWS_PER_W = 512
N_CHUNK = 8
CHUNK = 128


@functools.partial(
    pl.kernel,
    mesh=plsc.VectorSubcoreMesh(core_axis_name="c", subcore_axis_name="s"),
    out_type=jax.ShapeDtypeStruct((NW * ROWS_PER_W, 2 * EMBED), jnp.float32),
    scratch_types=[
        pltpu.VMEM((ROWS_PER_W, 2), jnp.float32),
        pltpu.VMEM((N_CHUNK, CHUNK), jnp.int32),
        pltpu.VMEM((2 * ROWS_PER_W, EMBED), jnp.float32),
        pltpu.SemaphoreType.DMA,
    ],
    compiler_params=pltpu.CompilerParams(
        use_tc_tiling_on_sc=False, needs_layout_passes=False),
)
def _sc_embed(x_hbm, table_hbm, out_hbm, xv, idxv, rows, sem):
    wid = lax.axis_index("s") * 2 + lax.axis_index("c")
    nb = wid * ROWS_PER_W

    pltpu.sync_copy(x_hbm.at[pl.ds(nb, ROWS_PER_W), :], xv)

    lanes = lax.iota(jnp.int32, 16)
    col0 = jnp.zeros((16,), jnp.int32)
    col1 = jnp.ones((16,), jnp.int32)
    for j in range(N_CHUNK // 2):
        def body(i, _):
            off = j * CHUNK + i * 16
            r16 = lanes + off
            e = plsc.load_gather(xv, [r16, col0])
            o = plsc.load_gather(xv, [r16, col1])
            idxv[j, pl.ds(i * 16, 16)] = (e * RES_F).astype(jnp.int32)
            idxv[j + N_CHUNK // 2, pl.ds(i * 16, 16)] = (
                o * RES_F).astype(jnp.int32)
            return 0
        lax.fori_loop(0, CHUNK // 16, body, 0)

    copies = []
    for j in range(N_CHUNK):
        copies.append(
            pltpu.async_copy(
                table_hbm.at[idxv.at[j]],
                rows.at[pl.ds(j * CHUNK, CHUNK)],
                sem,
            )
        )
    for c in copies:
        c.wait()

    pltpu.sync_copy(rows.at[pl.ds(0, ROWS_PER_W)],
                    out_hbm.at[pl.ds(nb, ROWS_PER_W), pl.ds(0, EMBED)])
    pltpu.sync_copy(rows.at[pl.ds(ROWS_PER_W, ROWS_PER_W)],
                    out_hbm.at[pl.ds(nb, ROWS_PER_W), pl.ds(EMBED, EMBED)])


def kernel(x, table):
    return _sc_embed(x, table)


# speedup vs baseline: 1.1073x; 1.1073x over previous
import jax
import jax.numpy as jnp
from jax import lax
from jax.experimental import pallas as pl
from jax.experimental.pallas import tpu as pltpu
from jax.experimental.pallas import tpu_sc as plsc
import functools

RES_F = 100000.0
EMBED = 64
NW = 32
ROWS_PER_W = 512
N_CHUNK = 8
CHUNK = 128

_MESH = plsc.VectorSubcoreMesh(core_axis_name="c", subcore_axis_name="s")


@functools.partial(
    pl.kernel,
    mesh=_MESH,
    out_type=jax.ShapeDtypeStruct((NW * N_CHUNK, CHUNK), jnp.int32),
    scratch_types=[
        pltpu.VMEM((ROWS_PER_W, 2), jnp.float32),
        pltpu.VMEM((N_CHUNK, CHUNK), jnp.int32),
    ],
    compiler_params=pltpu.CompilerParams(needs_layout_passes=False),
)
def _sc_indexify(x_hbm, idx_hbm, xv, idxv):
    wid = lax.axis_index("s") * 2 + lax.axis_index("c")
    nb = wid * ROWS_PER_W

    pltpu.sync_copy(x_hbm.at[pl.ds(nb, ROWS_PER_W), :], xv)

    lanes = lax.iota(jnp.int32, 16)
    col0 = jnp.zeros((16,), jnp.int32)
    col1 = jnp.ones((16,), jnp.int32)
    for j in range(N_CHUNK // 2):
        def body(i, _):
            off = j * CHUNK + i * 16
            r16 = lanes + off
            e = plsc.load_gather(xv, [r16, col0])
            o = plsc.load_gather(xv, [r16, col1])
            idxv[j, pl.ds(i * 16, 16)] = (e * RES_F).astype(jnp.int32)
            idxv[j + N_CHUNK // 2, pl.ds(i * 16, 16)] = (
                o * RES_F).astype(jnp.int32)
            return 0
        lax.fori_loop(0, CHUNK // 16, body, 0)

    pltpu.sync_copy(idxv, idx_hbm.at[pl.ds(wid * N_CHUNK, N_CHUNK)])


@functools.partial(
    pl.kernel,
    mesh=_MESH,
    out_type=jax.ShapeDtypeStruct((NW * ROWS_PER_W, 2 * EMBED), jnp.float32),
    scratch_types=[
        pltpu.VMEM((N_CHUNK, CHUNK), jnp.int32),
        pltpu.VMEM((2 * ROWS_PER_W, EMBED), jnp.float32),
        pltpu.SemaphoreType.DMA,
    ],
    compiler_params=pltpu.CompilerParams(
        use_tc_tiling_on_sc=False, needs_layout_passes=False),
)
def _sc_gather(idx_hbm, table_hbm, out_hbm, idxv, rows, sem):
    wid = lax.axis_index("s") * 2 + lax.axis_index("c")
    nb = wid * ROWS_PER_W

    pltpu.sync_copy(idx_hbm.at[pl.ds(wid * N_CHUNK, N_CHUNK)], idxv)

    copies = []
    for j in range(N_CHUNK):
        copies.append(
            pltpu.async_copy(
                table_hbm.at[idxv.at[j]],
                rows.at[pl.ds(j * CHUNK, CHUNK)],
                sem,
            )
        )
    for c in copies:
        c.wait()

    pltpu.sync_copy(rows.at[pl.ds(0, ROWS_PER_W)],
                    out_hbm.at[pl.ds(nb, ROWS_PER_W), pl.ds(0, EMBED)])
    pltpu.sync_copy(rows.at[pl.ds(ROWS_PER_W, ROWS_PER_W)],
                    out_hbm.at[pl.ds(nb, ROWS_PER_W), pl.ds(EMBED, EMBED)])


def kernel(x, table):
    idx = _sc_indexify(x)
    return _sc_gather(idx, table)
